# baseline (speedup 1.0000x reference)
import functools

import jax
import jax.numpy as jnp
from jax import lax
from jax.experimental import pallas as pl
from jax.experimental.pallas import tpu as pltpu
from jax.experimental.pallas import tpu_sc as plsc

D_MODEL = 1024
D_FF = 2048
N_EXP = 16
BT = 256
T_TOK = 2048
N_BLK = (T_TOK + N_EXP * BT) // BT
SLOTS = N_BLK * BT
NC, NS = 2, 16
NW = NC * NS
TPW = T_TOK // NW


def _router_body(x_ref, wr_ref, pos_ref, be_ref, bv_ref):
    x = x_ref[...]
    wr = wr_ref[...]
    logits = lax.dot_general(x, wr, (((1,), (1,)), ((), ())),
                             preferred_element_type=jnp.float32)
    rowmax = jnp.max(logits, axis=1, keepdims=True)
    e_iota = lax.broadcasted_iota(jnp.int32, (T_TOK, N_EXP), 1)
    eid = jnp.min(jnp.where(logits >= rowmax, e_iota, N_EXP),
                  axis=1, keepdims=True)
    onehot = (e_iota == eid).astype(jnp.float32)

    CH = 128
    c_r = lax.broadcasted_iota(jnp.int32, (CH, CH), 0)
    c_c = lax.broadcasted_iota(jnp.int32, (CH, CH), 1)
    tri_incl = (c_c <= c_r).astype(jnp.float32)
    ones_row = jnp.ones((1, CH), jnp.float32)
    pieces = []
    running = jnp.zeros((1, N_EXP), jnp.float32)
    for c in range(T_TOK // CH):
        chunk = onehot[c * CH:(c + 1) * CH]
        local = lax.dot_general(tri_incl, chunk, (((1,), (0,)), ((), ())),
                                preferred_element_type=jnp.float32)
        pieces.append(local + running)
        running = running + lax.dot_general(
            ones_row, chunk, (((1,), (0,)), ((), ())),
            preferred_element_type=jnp.float32)
    csum = jnp.concatenate(pieces, axis=0)
    counts = csum[T_TOK - 1:T_TOK, :]
    pc = jnp.ceil(counts * (1.0 / BT)) * BT
    e_r = lax.broadcasted_iota(jnp.int32, (N_EXP, N_EXP), 0)
    e_c = lax.broadcasted_iota(jnp.int32, (N_EXP, N_EXP), 1)
    excl = (e_r < e_c).astype(jnp.float32)
    incl = (e_r <= e_c).astype(jnp.float32)
    offs = lax.dot_general(pc, excl, (((1,), (0,)), ((), ())),
                           preferred_element_type=jnp.float32)
    cum = lax.dot_general(pc, incl, (((1,), (0,)), ((), ())),
                          preferred_element_type=jnp.float32)

    pos = jnp.sum(onehot * (csum - 1.0 + offs), axis=1)
    pos_ref[...] = pos.astype(jnp.int32)

    bstart = (lax.broadcasted_iota(jnp.int32, (N_BLK, N_EXP), 0)
              .astype(jnp.float32) * BT)
    be = jnp.sum((jnp.broadcast_to(cum, (N_BLK, N_EXP)) <= bstart)
                 .astype(jnp.int32), axis=1)
    be_ref[...] = jnp.minimum(be, N_EXP - 1)
    total = jnp.sum(pc)
    bv_ref[...] = (bstart[:, 0] < total).astype(jnp.int32)


def _ffn_body(be_ref, bv_ref, x_ref, w1_ref, b1_ref, w2_ref, b2_ref, o_ref):
    b = pl.program_id(0)

    @pl.when(bv_ref[b] == 1)
    def _():
        xb = x_ref[...]
        h = lax.dot_general(xb, w1_ref[0], (((1,), (0,)), ((), ())),
                            preferred_element_type=jnp.float32) + b1_ref[0]
        h = 0.5 * h * (1.0 + lax.erf(h * 0.7071067811865476))
        y = lax.dot_general(h, w2_ref[0], (((1,), (0,)), ((), ())),
                            preferred_element_type=jnp.float32) + b2_ref[0]
        o_ref[...] = y


def _router_call(xf, Wr):
    return pl.pallas_call(
        _router_body,
        out_shape=(
            jax.ShapeDtypeStruct((T_TOK,), jnp.int32),
            jax.ShapeDtypeStruct((N_BLK,), jnp.int32),
            jax.ShapeDtypeStruct((N_BLK,), jnp.int32),
        ),
    )(xf, Wr)


def _ffn_call(be, bv, xs, W1, b1, W2, b2):
    grid_spec = pltpu.PrefetchScalarGridSpec(
        num_scalar_prefetch=2,
        grid=(N_BLK,),
        in_specs=[
            pl.BlockSpec((BT, D_MODEL), lambda b, be, bv: (b, 0)),
            pl.BlockSpec((1, D_MODEL, D_FF), lambda b, be, bv: (be[b], 0, 0)),
            pl.BlockSpec((1, 1, D_FF), lambda b, be, bv: (be[b], 0, 0)),
            pl.BlockSpec((1, D_FF, D_MODEL), lambda b, be, bv: (be[b], 0, 0)),
            pl.BlockSpec((1, 1, D_MODEL), lambda b, be, bv: (be[b], 0, 0)),
        ],
        out_specs=pl.BlockSpec((BT, D_MODEL), lambda b, be, bv: (b, 0)),
    )
    return pl.pallas_call(
        _ffn_body,
        grid_spec=grid_spec,
        out_shape=jax.ShapeDtypeStruct((SLOTS, D_MODEL), jnp.float32),
    )(be, bv, xs, W1, b1.reshape(N_EXP, 1, D_FF), W2,
      b2.reshape(N_EXP, 1, D_MODEL))


def _dispatch_call(xf, pos):
    mesh = plsc.VectorSubcoreMesh(core_axis_name="c", subcore_axis_name="s")

    @functools.partial(
        pl.kernel, mesh=mesh,
        out_type=jax.ShapeDtypeStruct((SLOTS, D_MODEL), jnp.float32),
        scratch_types=[
            pltpu.VMEM((TPW,), jnp.int32),
            pltpu.VMEM((TPW, D_MODEL), jnp.float32),
            pltpu.SemaphoreType.DMA,
        ],
    )
    def scatter_k(x_hbm, pos_hbm, xs_hbm, idx_v, rows_v, sem):
        wid = lax.axis_index("s") * NC + lax.axis_index("c")
        base = wid * TPW
        pltpu.sync_copy(pos_hbm.at[pl.ds(base, TPW)], idx_v)
        pltpu.sync_copy(x_hbm.at[pl.ds(base, TPW)], rows_v)
        pltpu.async_copy(rows_v, xs_hbm.at[idx_v], sem).wait()

    return scatter_k(xf, pos)


def _combine_call(ys, pos):
    mesh = plsc.VectorSubcoreMesh(core_axis_name="c", subcore_axis_name="s")

    @functools.partial(
        pl.kernel, mesh=mesh,
        out_type=jax.ShapeDtypeStruct((T_TOK, D_MODEL), jnp.float32),
        scratch_types=[
            pltpu.VMEM((TPW,), jnp.int32),
            pltpu.VMEM((TPW, D_MODEL), jnp.float32),
            pltpu.SemaphoreType.DMA,
        ],
    )
    def gather_k(ys_hbm, pos_hbm, out_hbm, idx_v, rows_v, sem):
        wid = lax.axis_index("s") * NC + lax.axis_index("c")
        base = wid * TPW
        pltpu.sync_copy(pos_hbm.at[pl.ds(base, TPW)], idx_v)
        pltpu.async_copy(ys_hbm.at[idx_v], rows_v, sem).wait()
        pltpu.sync_copy(rows_v, out_hbm.at[pl.ds(base, TPW)])

    return gather_k(ys, pos)


def kernel(x, Wr, W1, b1, W2, b2):
    B, T, D = x.shape
    xf = x.reshape(T, D)
    pos, be, bv = _router_call(xf, Wr)
    xs = _dispatch_call(xf, pos)
    ys = _ffn_call(be, bv, xs, W1, b1, W2, b2)
    out = _combine_call(ys, pos)
    return out.reshape(B, T, D)
